# Initial kernel scaffold; baseline (speedup 1.0000x reference)
import jax
import jax.numpy as jnp
from jax.experimental import pallas as pl


def kernel(t, beta, alpha):
    raise NotImplementedError("write your pallas kernel here")


# speedup vs baseline: 8.3196x; 8.3196x over previous
import functools

import jax
import jax.numpy as jnp
from jax import lax
from jax.experimental import pallas as pl
from jax.experimental.pallas import tpu as pltpu
from jax.experimental.pallas import tpu_sc as plsc

_BATCH = 16384
_TABLE = 1000
_NC = 2
_NS = 16
_L = 16
_NW = _NC * _NS
_B_PER_W = _BATCH // _NW


@functools.partial(
    pl.kernel,
    mesh=plsc.VectorSubcoreMesh(core_axis_name="c", subcore_axis_name="s"),
    out_type=(
        jax.ShapeDtypeStruct((_BATCH,), jnp.float32),
        jax.ShapeDtypeStruct((_BATCH,), jnp.float32),
    ),
    scratch_types=[
        pltpu.VMEM((_B_PER_W,), jnp.int32),
        pltpu.VMEM((_TABLE,), jnp.float32),
        pltpu.VMEM((_TABLE,), jnp.float32),
        pltpu.VMEM((_B_PER_W,), jnp.float32),
        pltpu.VMEM((_B_PER_W,), jnp.float32),
    ],
    compiler_params=pltpu.CompilerParams(needs_layout_passes=False),
)
def _gather_sc(t_hbm, beta_hbm, alpha_hbm, bt_hbm, at_hbm,
               idx_v, beta_v, alpha_v, bt_v, at_v):
    wid = lax.axis_index("s") * _NC + lax.axis_index("c")
    base = wid * _B_PER_W
    pltpu.sync_copy(beta_hbm, beta_v)
    pltpu.sync_copy(alpha_hbm, alpha_v)
    pltpu.sync_copy(t_hbm.at[pl.ds(base, _B_PER_W)], idx_v)
    for i in range(_B_PER_W // _L):
        idx = idx_v[pl.ds(i * _L, _L)]
        bt_v[pl.ds(i * _L, _L)] = plsc.load_gather(beta_v, [idx])
        at_v[pl.ds(i * _L, _L)] = plsc.load_gather(alpha_v, [idx])
    pltpu.sync_copy(bt_v, bt_hbm.at[pl.ds(base, _B_PER_W)])
    pltpu.sync_copy(at_v, at_hbm.at[pl.ds(base, _B_PER_W)])


def kernel(t, beta, alpha):
    return _gather_sc(t.astype(jnp.int32), beta, alpha)


# speedup vs baseline: 8.8321x; 1.0616x over previous
import functools

import jax
import jax.numpy as jnp
from jax import lax
from jax.experimental import pallas as pl
from jax.experimental.pallas import tpu as pltpu
from jax.experimental.pallas import tpu_sc as plsc

_BATCH = 16384
_TABLE = 1000
_NC = 2
_NS = 16
_L = 16
_NW = _NC * _NS
_B_PER_W = _BATCH // _NW


@functools.partial(
    pl.kernel,
    mesh=plsc.VectorSubcoreMesh(core_axis_name="c", subcore_axis_name="s"),
    out_type=(
        jax.ShapeDtypeStruct((_BATCH,), jnp.float32),
        jax.ShapeDtypeStruct((_BATCH,), jnp.float32),
    ),
    scratch_types=[
        pltpu.VMEM((_B_PER_W,), jnp.int32),
        pltpu.VMEM((_TABLE,), jnp.float32),
        pltpu.VMEM((_TABLE,), jnp.float32),
        pltpu.VMEM((_B_PER_W,), jnp.float32),
        pltpu.VMEM((_B_PER_W,), jnp.float32),
        pltpu.SemaphoreType.DMA,
    ],
    compiler_params=pltpu.CompilerParams(needs_layout_passes=False),
)
def _gather_sc(t_hbm, beta_hbm, alpha_hbm, bt_hbm, at_hbm,
               idx_v, beta_v, alpha_v, bt_v, at_v, sem):
    wid = lax.axis_index("s") * _NC + lax.axis_index("c")
    base = wid * _B_PER_W
    c1 = pltpu.async_copy(beta_hbm, beta_v, sem)
    c2 = pltpu.async_copy(alpha_hbm, alpha_v, sem)
    c3 = pltpu.async_copy(t_hbm.at[pl.ds(base, _B_PER_W)], idx_v, sem)
    c1.wait()
    c2.wait()
    c3.wait()
    for i in range(_B_PER_W // _L):
        idx = idx_v[pl.ds(i * _L, _L)]
        bt_v[pl.ds(i * _L, _L)] = plsc.load_gather(beta_v, [idx])
        at_v[pl.ds(i * _L, _L)] = plsc.load_gather(alpha_v, [idx])
    o1 = pltpu.async_copy(bt_v, bt_hbm.at[pl.ds(base, _B_PER_W)], sem)
    o2 = pltpu.async_copy(at_v, at_hbm.at[pl.ds(base, _B_PER_W)], sem)
    o1.wait()
    o2.wait()


def kernel(t, beta, alpha):
    return _gather_sc(t.astype(jnp.int32), beta, alpha)


# speedup vs baseline: 9.2434x; 1.0466x over previous
import functools

import jax
import jax.numpy as jnp
from jax import lax
from jax.experimental import pallas as pl
from jax.experimental.pallas import tpu as pltpu
from jax.experimental.pallas import tpu_sc as plsc

_BATCH = 16384
_TABLE = 1000
_NC = 1
_NS = 16
_L = 16
_NW = _NC * _NS
_B_PER_W = _BATCH // _NW


@functools.partial(
    pl.kernel,
    mesh=plsc.VectorSubcoreMesh(
        core_axis_name="c", subcore_axis_name="s", num_cores=_NC),
    out_type=(
        jax.ShapeDtypeStruct((_BATCH,), jnp.float32),
        jax.ShapeDtypeStruct((_BATCH,), jnp.float32),
    ),
    scratch_types=[
        pltpu.VMEM((_B_PER_W,), jnp.int32),
        pltpu.VMEM((_TABLE,), jnp.float32),
        pltpu.VMEM((_TABLE,), jnp.float32),
        pltpu.VMEM((_B_PER_W,), jnp.float32),
        pltpu.VMEM((_B_PER_W,), jnp.float32),
        pltpu.SemaphoreType.DMA,
    ],
    compiler_params=pltpu.CompilerParams(needs_layout_passes=False),
)
def _gather_sc(t_hbm, beta_hbm, alpha_hbm, bt_hbm, at_hbm,
               idx_v, beta_v, alpha_v, bt_v, at_v, sem):
    wid = lax.axis_index("s") * _NC + lax.axis_index("c")
    base = wid * _B_PER_W
    c1 = pltpu.async_copy(beta_hbm, beta_v, sem)
    c2 = pltpu.async_copy(alpha_hbm, alpha_v, sem)
    c3 = pltpu.async_copy(t_hbm.at[pl.ds(base, _B_PER_W)], idx_v, sem)
    c1.wait()
    c2.wait()
    c3.wait()
    for i in range(_B_PER_W // _L):
        idx = idx_v[pl.ds(i * _L, _L)]
        bt_v[pl.ds(i * _L, _L)] = plsc.load_gather(beta_v, [idx])
        at_v[pl.ds(i * _L, _L)] = plsc.load_gather(alpha_v, [idx])
    o1 = pltpu.async_copy(bt_v, bt_hbm.at[pl.ds(base, _B_PER_W)], sem)
    o2 = pltpu.async_copy(at_v, at_hbm.at[pl.ds(base, _B_PER_W)], sem)
    o1.wait()
    o2.wait()


def kernel(t, beta, alpha):
    return _gather_sc(t.astype(jnp.int32), beta, alpha)


# speedup vs baseline: 9.6913x; 1.0485x over previous
import functools

import jax
import jax.numpy as jnp
from jax import lax
from jax.experimental import pallas as pl
from jax.experimental.pallas import tpu as pltpu
from jax.experimental.pallas import tpu_sc as plsc

_BATCH = 16384
_TABLE = 1000
_NC = 1
_NS = 16
_L = 16
_NW = _NC * _NS
_B_PER_W = _BATCH // _NW


@functools.partial(
    pl.kernel,
    mesh=plsc.VectorSubcoreMesh(
        core_axis_name="c", subcore_axis_name="s", num_cores=_NC),
    out_type=(
        jax.ShapeDtypeStruct((_BATCH,), jnp.float32),
        jax.ShapeDtypeStruct((_BATCH,), jnp.float32),
    ),
    scratch_types=[
        pltpu.VMEM((_B_PER_W,), jnp.int32),
        pltpu.VMEM((_TABLE,), jnp.float32),
        pltpu.VMEM((_TABLE,), jnp.float32),
        pltpu.VMEM((_B_PER_W,), jnp.float32),
        pltpu.VMEM((_B_PER_W,), jnp.float32),
        pltpu.SemaphoreType.DMA,
    ],
    compiler_params=pltpu.CompilerParams(needs_layout_passes=False),
)
def _gather_sc(t_hbm, beta_hbm, alpha_hbm, bt_hbm, at_hbm,
               idx_v, beta_v, alpha_v, bt_v, at_v, sem):
    wid = lax.axis_index("s") * _NC + lax.axis_index("c")
    base = wid * _B_PER_W
    c1 = pltpu.async_copy(beta_hbm, beta_v, sem)
    c2 = pltpu.async_copy(alpha_hbm, alpha_v, sem)
    c3 = pltpu.async_copy(t_hbm.at[pl.ds(base, _B_PER_W)], idx_v, sem)
    c1.wait()
    c2.wait()
    c3.wait()
    @plsc.parallel_loop(0, _B_PER_W, step=_L, unroll=4)
    def _(i):
        idx = idx_v[pl.ds(i, _L)]
        bt_v[pl.ds(i, _L)] = plsc.load_gather(beta_v, [idx])
        at_v[pl.ds(i, _L)] = plsc.load_gather(alpha_v, [idx])
    o1 = pltpu.async_copy(bt_v, bt_hbm.at[pl.ds(base, _B_PER_W)], sem)
    o2 = pltpu.async_copy(at_v, at_hbm.at[pl.ds(base, _B_PER_W)], sem)
    o1.wait()
    o2.wait()


def kernel(t, beta, alpha):
    return _gather_sc(t.astype(jnp.int32), beta, alpha)
